# baseline (speedup 1.0000x reference)
import functools

import jax
import jax.numpy as jnp
from jax import lax
from jax.experimental import pallas as pl
from jax.experimental.pallas import tpu as pltpu
from jax.experimental.pallas import tpu_sc as plsc

_NC = 2
_NS = 16
_NW = _NC * _NS
_LANES = 16
_CHUNK_ROWS = 8
_NBUF = 4


def _sc_add(x, pos):
    batch, seq_len, dim = x.shape
    total_rows = batch * seq_len
    rows_per_w = total_rows // _NW
    n_chunks = rows_per_w // _CHUNK_ROWS
    chunk_words = _CHUNK_ROWS * dim
    n_outer = n_chunks // _NBUF
    x1 = x.reshape(-1)
    p1 = pos.reshape(-1)

    mesh = plsc.VectorSubcoreMesh(
        core_axis_name="c", subcore_axis_name="s",
        num_cores=_NC, num_subcores=_NS,
    )

    vmem = lambda: pltpu.VMEM((chunk_words,), jnp.float32)

    @functools.partial(
        pl.kernel,
        out_type=jax.ShapeDtypeStruct((total_rows * dim,), jnp.float32),
        mesh=mesh,
        scratch_types=(
            [vmem() for _ in range(3 * _NBUF)]
            + [pltpu.SemaphoreType.DMA for _ in range(3 * _NBUF)]
        ),
    )
    def k(x_hbm, p_hbm, o_hbm, *scr):
        xbufs = scr[0:_NBUF]
        pbufs = scr[_NBUF:2 * _NBUF]
        obufs = scr[2 * _NBUF:3 * _NBUF]
        sxs = scr[3 * _NBUF:4 * _NBUF]
        sps = scr[4 * _NBUF:5 * _NBUF]
        sos = scr[5 * _NBUF:6 * _NBUF]
        wid = lax.axis_index("s") * _NC + lax.axis_index("c")
        xbase = wid * (rows_per_w * dim)
        pbase = ((wid * rows_per_w) % seq_len) * dim

        def in_copies(c, b):
            off = c * chunk_words
            return (
                pltpu.make_async_copy(
                    x_hbm.at[pl.ds(xbase + off, chunk_words)], xbufs[b], sxs[b]),
                pltpu.make_async_copy(
                    p_hbm.at[pl.ds(pbase + off, chunk_words)], pbufs[b], sps[b]),
            )

        def out_copy(c, b):
            off = c * chunk_words
            return pltpu.make_async_copy(
                obufs[b], o_hbm.at[pl.ds(xbase + off, chunk_words)], sos[b])

        for b in range(_NBUF):
            for cp in in_copies(b, b):
                cp.start()

        def outer(g, carry):
            for b in range(_NBUF):
                c = g * _NBUF + b
                for cp in in_copies(c, b):
                    cp.wait()

                @pl.when(g >= 1)
                def _():
                    out_copy(c - _NBUF, b).wait()

                ob, xbuf, pbuf = obufs[b], xbufs[b], pbufs[b]

                @plsc.parallel_loop(0, chunk_words // _LANES, unroll=8)
                def _(i):
                    sl = pl.ds(i * _LANES, _LANES)
                    ob[sl] = xbuf[sl] + pbuf[sl]

                out_copy(c, b).start()

                @pl.when(g < n_outer - 1)
                def _():
                    for cp in in_copies(c + _NBUF, b):
                        cp.start()
            return carry

        lax.fori_loop(0, n_outer, outer, 0)
        for b in range(_NBUF):
            out_copy(n_chunks - _NBUF + b, b).wait()

    return k(x1, p1).reshape(x.shape)


def kernel(x, pos_table):
    seq_len = x.shape[1]
    return _sc_add(x, pos_table[:seq_len])


# speedup vs baseline: 4.4787x; 4.4787x over previous
import jax
import jax.numpy as jnp
from jax.experimental import pallas as pl

BATCH_BLOCK = 2
SEQ_BLOCK = 1024


def _tc_body(x_ref, pos_ref, out_ref):
    out_ref[...] = x_ref[...] + pos_ref[...][None, :, :]


def kernel(x, pos_table):
    batch, seq_len, dim = x.shape
    bb = BATCH_BLOCK if batch % BATCH_BLOCK == 0 else 1
    sb = SEQ_BLOCK if seq_len % SEQ_BLOCK == 0 else seq_len
    grid = (seq_len // sb, batch // bb)
    return pl.pallas_call(
        _tc_body,
        grid=grid,
        in_specs=[
            pl.BlockSpec((bb, sb, dim), lambda i, j: (j, i, 0)),
            pl.BlockSpec((sb, dim), lambda i, j: (i, 0)),
        ],
        out_specs=pl.BlockSpec((bb, sb, dim), lambda i, j: (j, i, 0)),
        out_shape=jax.ShapeDtypeStruct(x.shape, x.dtype),
    )(x, pos_table[:seq_len])
